# baseline (device time: 62305 ns/iter reference)
import jax
import jax.numpy as jnp
from jax import lax
from jax.experimental import pallas as pl
from jax.experimental.pallas import tpu as pltpu

N_DEV = 4


def kernel(x, Wg, Wu, Wd):
    m, _ = x.shape
    d = Wd.shape[1]
    c = m // N_DEV

    xb = x.astype(jnp.bfloat16)
    Wgb = Wg.astype(jnp.bfloat16)
    Wub = Wu.astype(jnp.bfloat16)
    Wdb = Wd.astype(jnp.bfloat16)

    def body(x_ref, wg_ref, wu_ref, wd_ref, out_ref,
             rs_send, rs_buf,
             rs_send_sems, rs_recv_sems, ag_send_sems, ag_recv_sems):
        my = lax.axis_index("i")

        barrier_sem = pltpu.get_barrier_semaphore()
        for off in (1, 2, 3):
            pl.semaphore_signal(
                barrier_sem, inc=1,
                device_id=((my + off) % N_DEV,),
                device_id_type=pl.DeviceIdType.MESH,
            )
        pl.semaphore_wait(barrier_sem, 3)

        wd = wd_ref[...]

        def chunk_partial(j):
            xj = x_ref[pl.ds(j * c, c), :]
            gj = jnp.dot(xj, wg_ref[...], preferred_element_type=jnp.float32)
            uj = jnp.dot(xj, wu_ref[...], preferred_element_type=jnp.float32)
            hj = (gj * (uj * jax.nn.sigmoid(uj))).astype(jnp.bfloat16)
            return jnp.dot(hj, wd, preferred_element_type=jnp.float32)

        rs_rdmas = {}
        for off in (1, 2, 3):
            j = (my + off) % N_DEV
            rs_send[off - 1] = chunk_partial(j).astype(jnp.bfloat16)
            rdma = pltpu.make_async_remote_copy(
                src_ref=rs_send.at[off - 1],
                dst_ref=rs_buf.at[off - 1],
                send_sem=rs_send_sems.at[off - 1],
                recv_sem=rs_recv_sems.at[off - 1],
                device_id=(j,),
                device_id_type=pl.DeviceIdType.MESH,
            )
            rdma.start()
            rs_rdmas[off - 1] = rdma

        acc = chunk_partial(my)

        for k in range(3):
            rs_rdmas[k].wait_recv()
            acc = acc + rs_buf[k].astype(jnp.float32)
        out_ref[pl.ds(my * c, c), :] = acc.astype(jnp.bfloat16)

        ag_rdmas = []
        for off in (1, 2, 3):
            j = (my + off) % N_DEV
            rdma = pltpu.make_async_remote_copy(
                src_ref=out_ref.at[pl.ds(my * c, c), :],
                dst_ref=out_ref.at[pl.ds(my * c, c), :],
                send_sem=ag_send_sems.at[off - 1],
                recv_sem=ag_recv_sems.at[off - 1],
                device_id=(j,),
                device_id_type=pl.DeviceIdType.MESH,
            )
            rdma.start()
            ag_rdmas.append(rdma)

        for r in ag_rdmas:
            r.wait_recv()
        for r in list(rs_rdmas.values()) + ag_rdmas:
            r.wait_send()

    return pl.pallas_call(
        body,
        out_shape=jax.ShapeDtypeStruct((m, d), jnp.bfloat16),
        in_specs=[pl.BlockSpec(memory_space=pltpu.VMEM)] * 4,
        out_specs=pl.BlockSpec(memory_space=pltpu.VMEM),
        scratch_shapes=[
            pltpu.VMEM((3, c, d), jnp.bfloat16),
            pltpu.VMEM((3, c, d), jnp.bfloat16),
            pltpu.SemaphoreType.DMA((3,)),
            pltpu.SemaphoreType.DMA((3,)),
            pltpu.SemaphoreType.DMA((3,)),
            pltpu.SemaphoreType.DMA((3,)),
        ],
        compiler_params=pltpu.CompilerParams(
            collective_id=0, vmem_limit_bytes=100 * 1024 * 1024,
        ),
    )(xb, Wgb, Wub, Wdb)


# device time: 48492 ns/iter; 1.2849x vs baseline; 1.2849x over previous
import jax
import jax.numpy as jnp
from jax import lax
from jax.experimental import pallas as pl
from jax.experimental.pallas import tpu as pltpu

N_DEV = 4


def kernel(x, Wg, Wu, Wd):
    m, _ = x.shape
    d = Wd.shape[1]
    c = m // N_DEV

    xb = x.astype(jnp.bfloat16)
    Wgb = Wg.astype(jnp.bfloat16)
    Wub = Wu.astype(jnp.bfloat16)
    Wdb = Wd.astype(jnp.bfloat16)

    def body(x_ref, wg_ref, wu_ref, wd_ref, out_ref,
             rs_send, rs_buf,
             rs_send_sems, rs_recv_sems, ag_send_sems, ag_recv_sems):
        my = lax.axis_index("i")

        barrier_sem = pltpu.get_barrier_semaphore()
        for off in (1, 2, 3):
            pl.semaphore_signal(
                barrier_sem, inc=1,
                device_id=((my + off) % N_DEV,),
                device_id_type=pl.DeviceIdType.MESH,
            )
        pl.semaphore_wait(barrier_sem, 3)

        wd = wd_ref[...]

        def chunk_partial(j):
            xj = x_ref[pl.ds(j * c, c), :]
            gj = jnp.dot(xj, wg_ref[...], preferred_element_type=jnp.float32)
            uj = jnp.dot(xj, wu_ref[...], preferred_element_type=jnp.float32)
            hj = (gj * (uj * jax.nn.sigmoid(uj))).astype(jnp.bfloat16)
            return jnp.dot(hj, wd, preferred_element_type=jnp.float32)

        rs_rdmas = {}
        for off in (1, 2, 3):
            j = (my + off) % N_DEV
            rs_send[off - 1] = chunk_partial(j).astype(jnp.bfloat16)
            rdma = pltpu.make_async_remote_copy(
                src_ref=rs_send.at[off - 1],
                dst_ref=rs_buf.at[off - 1],
                send_sem=rs_send_sems.at[off - 1],
                recv_sem=rs_recv_sems.at[off - 1],
                device_id=(j,),
                device_id_type=pl.DeviceIdType.MESH,
            )
            rdma.start()
            rs_rdmas[off - 1] = rdma

        acc = chunk_partial(my)

        for k in range(3):
            rs_rdmas[k].wait_recv()
            acc = acc + rs_buf[k].astype(jnp.float32)
        out_ref[pl.ds(my * c, c), :] = acc.astype(jnp.bfloat16)

        ag_rdmas = []
        for off in (1, 2, 3):
            j = (my + off) % N_DEV
            rdma = pltpu.make_async_remote_copy(
                src_ref=out_ref.at[pl.ds(my * c, c), :],
                dst_ref=out_ref.at[pl.ds(my * c, c), :],
                send_sem=ag_send_sems.at[off - 1],
                recv_sem=ag_recv_sems.at[off - 1],
                device_id=(j,),
                device_id_type=pl.DeviceIdType.MESH,
            )
            rdma.start()
            ag_rdmas.append(rdma)

        for r in ag_rdmas:
            r.wait_recv()
        for r in list(rs_rdmas.values()) + ag_rdmas:
            r.wait_send()

    return pl.pallas_call(
        body,
        out_shape=jax.ShapeDtypeStruct((m, d), jnp.bfloat16),
        in_specs=[pl.BlockSpec(memory_space=pltpu.VMEM)] * 4,
        out_specs=pl.BlockSpec(memory_space=pltpu.VMEM),
        scratch_shapes=[
            pltpu.VMEM((3, c, d), jnp.bfloat16),
            pltpu.VMEM((3, c, d), jnp.bfloat16),
            pltpu.SemaphoreType.DMA((3,)),
            pltpu.SemaphoreType.DMA((3,)),
            pltpu.SemaphoreType.DMA((3,)),
            pltpu.SemaphoreType.DMA((3,)),
        ],
        compiler_params=pltpu.CompilerParams(collective_id=0),
    )(xb, Wgb, Wub, Wdb)


# device time: 45393 ns/iter; 1.3726x vs baseline; 1.0683x over previous
import jax
import jax.numpy as jnp
from jax import lax
from jax.experimental import pallas as pl
from jax.experimental.pallas import tpu as pltpu

N_DEV = 4


def kernel(x, Wg, Wu, Wd):
    m, _ = x.shape
    d = Wd.shape[1]
    c = m // N_DEV
    hc = c // 2

    xb = x.astype(jnp.bfloat16)
    Wgb = Wg.astype(jnp.bfloat16)
    Wub = Wu.astype(jnp.bfloat16)
    Wdb = Wd.astype(jnp.bfloat16)

    def body(x_ref, wg_ref, wu_ref, wd_ref, out_ref,
             rs_send, rs_buf,
             rs_send_sems, rs_recv_sems, ag_send_sems, ag_recv_sems):
        my = lax.axis_index("i")

        barrier_sem = pltpu.get_barrier_semaphore()
        for off in (1, 2, 3):
            pl.semaphore_signal(
                barrier_sem, inc=1,
                device_id=((my + off) % N_DEV,),
                device_id_type=pl.DeviceIdType.MESH,
            )
        pl.semaphore_wait(barrier_sem, 3)

        def partial_rows(r0):
            xj = x_ref[pl.ds(r0, hc), :]
            gj = jnp.dot(xj, wg_ref[...], preferred_element_type=jnp.float32)
            uj = jnp.dot(xj, wu_ref[...], preferred_element_type=jnp.float32)
            hj = (gj * (uj * jax.nn.sigmoid(uj))).astype(jnp.bfloat16)
            return jnp.dot(hj, wd_ref[...], preferred_element_type=jnp.float32)

        rs_rdmas = {}

        def compute_and_send(off, half):
            j = (my + off) % N_DEV
            slot = (off - 1) * 2 + half
            rs_send[slot] = partial_rows(j * c + half * hc).astype(jnp.bfloat16)
            rdma = pltpu.make_async_remote_copy(
                src_ref=rs_send.at[slot],
                dst_ref=rs_buf.at[slot],
                send_sem=rs_send_sems.at[slot],
                recv_sem=rs_recv_sems.at[slot],
                device_id=(j,),
                device_id_type=pl.DeviceIdType.MESH,
            )
            rdma.start()
            rs_rdmas[slot] = rdma

        ag_rdmas = []

        def reduce_and_broadcast(half, acc):
            for off in (1, 2, 3):
                slot = (off - 1) * 2 + half
                rs_rdmas[slot].wait_recv()
                acc = acc + rs_buf[slot].astype(jnp.float32)
            r0 = my * c + half * hc
            out_ref[pl.ds(r0, hc), :] = acc.astype(jnp.bfloat16)
            for off in (1, 2, 3):
                j = (my + off) % N_DEV
                slot = (off - 1) * 2 + half
                rdma = pltpu.make_async_remote_copy(
                    src_ref=out_ref.at[pl.ds(r0, hc), :],
                    dst_ref=out_ref.at[pl.ds(r0, hc), :],
                    send_sem=ag_send_sems.at[slot],
                    recv_sem=ag_recv_sems.at[slot],
                    device_id=(j,),
                    device_id_type=pl.DeviceIdType.MESH,
                )
                rdma.start()
                ag_rdmas.append(rdma)

        for off in (1, 2, 3):
            compute_and_send(off, 0)
        acc_a = partial_rows(my * c)
        for off in (1, 2, 3):
            compute_and_send(off, 1)
        reduce_and_broadcast(0, acc_a)
        acc_b = partial_rows(my * c + hc)
        reduce_and_broadcast(1, acc_b)

        for r in ag_rdmas:
            r.wait_recv()
        for r in list(rs_rdmas.values()) + ag_rdmas:
            r.wait_send()

    return pl.pallas_call(
        body,
        out_shape=jax.ShapeDtypeStruct((m, d), jnp.bfloat16),
        in_specs=[pl.BlockSpec(memory_space=pltpu.VMEM)] * 4,
        out_specs=pl.BlockSpec(memory_space=pltpu.VMEM),
        scratch_shapes=[
            pltpu.VMEM((6, hc, d), jnp.bfloat16),
            pltpu.VMEM((6, hc, d), jnp.bfloat16),
            pltpu.SemaphoreType.DMA((6,)),
            pltpu.SemaphoreType.DMA((6,)),
            pltpu.SemaphoreType.DMA((6,)),
            pltpu.SemaphoreType.DMA((6,)),
        ],
        compiler_params=pltpu.CompilerParams(collective_id=0),
    )(xb, Wgb, Wub, Wdb)
